# baseline (speedup 1.0000x reference)
import jax
import jax.numpy as jnp
from jax import lax
from jax.experimental import pallas as pl
from jax.experimental.pallas import tpu as pltpu
from jax.experimental.pallas import tpu_sc as plsc

_VOCAB = 100000
_EMBED = 64
_B = 4096
_L = 200

_NC = 2
_NS = 16
_NW = _NC * _NS
_LANES = 16

_N = _B * _L
_RPW = _N // _NW
_CH = 128
_NCHUNK = _RPW // _CH
_NSLOT = 3
_NWIN = (2 * _CH) // 128


def _sc_body(x_hbm, table_hbm, gamma_hbm, beta_hbm, out_hbm, mask_hbm,
             xt_v, idx_v, rows_v, mask_v, gb_v, sem_g, sem_o):
    wid = lax.axis_index("s") * _NC + lax.axis_index("c")

    pltpu.sync_copy(gamma_hbm, gb_v.at[0])
    pltpu.sync_copy(beta_hbm, gb_v.at[1])
    g = [gb_v[0, pl.ds(16 * p, 16)] for p in range(4)]
    b = [gb_v[1, pl.ds(16 * p, 16)] for p in range(4)]

    one = jnp.full((_LANES,), 1, dtype=jnp.int32)
    zero = jnp.full((_LANES,), 0, dtype=jnp.int32)
    lane = lax.iota(jnp.int32, _LANES)
    evens = lane * 2

    perms = [lax.bitwise_xor(lane, jnp.int32(k)) for k in (8, 4, 2, 1)]
    dnums = lax.GatherDimensionNumbers(
        offset_dims=(), collapsed_slice_dims=(0,), start_index_map=(0,))

    def allsum(vec):
        for p in perms:
            vec = vec + lax.gather(
                vec, p[:, None], dnums, slice_sizes=(1,),
                mode=lax.GatherScatterMode.PROMISE_IN_BOUNDS)
        return vec

    def rsqrt_vec(x):
        i = lax.bitcast_convert_type(x, jnp.int32)
        i = jnp.int32(0x5F3759DF) - lax.shift_right_logical(i, 1)
        y = lax.bitcast_convert_type(i, jnp.float32)
        xh = x * jnp.float32(0.5)
        for _ in range(2):
            y = y * (jnp.float32(1.5) - xh * y * y)
        return y

    def fire_chunk(c, s):
        fbase = wid * _RPW + c * _CH
        pltpu.sync_copy(x_hbm.at[pl.ds(fbase, _CH)], xt_v.at[s])
        islot = idx_v.at[s]
        for t in range(_CH // 16):
            iv = xt_v[s, pl.ds(16 * t, 16)]
            pos = evens + (32 * t)
            plsc.store_scatter(islot, [pos], iv)
            plsc.store_scatter(islot, [pos + 1], iv)
        for w in range(_NWIN):
            pltpu.async_copy(table_hbm.at[idx_v.at[s, pl.ds(w * 128, 128)]],
                             rows_v.at[s, pl.ds(w * 128, 128)], sem_g.at[s])

    def drain_gathers(s):
        for w in range(_NWIN):
            pltpu.make_async_copy(
                table_hbm.at[idx_v.at[s, pl.ds(w * 128, 128)]],
                rows_v.at[s, pl.ds(w * 128, 128)], sem_g.at[s]).wait()

    def fire_out(c, s):
        base = wid * _RPW + c * _CH
        pltpu.async_copy(rows_v.at[s], out_hbm.at[pl.ds(2 * base, 2 * _CH)],
                         sem_o.at[s])
        pltpu.async_copy(mask_v.at[s], mask_hbm.at[pl.ds(base, _CH)],
                         sem_o.at[s])

    def drain_out(s):
        pltpu.make_async_copy(rows_v.at[s], out_hbm.at[pl.ds(0, 2 * _CH)],
                              sem_o.at[s]).wait()
        pltpu.make_async_copy(mask_v.at[s], mask_hbm.at[pl.ds(0, _CH)],
                              sem_o.at[s]).wait()

    def compute_chunk(s):
        for t in range(_CH // 16):
            iv = xt_v[s, pl.ds(16 * t, 16)]
            mask_v[s, pl.ds(16 * t, 16)] = jnp.where(iv > 0, one, zero)

        rslot = rows_v.at[s]

        def row_body(r, carry):
            r2 = r * 2
            v = [rslot[r2, pl.ds(16 * p, 16)] for p in range(4)]
            ssum = (v[0] + v[1]) + (v[2] + v[3])
            sq = (v[0] * v[0] + v[1] * v[1]) + (v[2] * v[2] + v[3] * v[3])
            mean = allsum(ssum) * jnp.float32(1.0 / 64.0)
            var = allsum(sq) * jnp.float32(1.0 / 64.0) - mean * mean
            inv = rsqrt_vec(var + jnp.float32(1e-5))
            for p in range(4):
                rslot[r2, pl.ds(16 * p, 16)] = \
                    (v[p] - mean) * (inv * g[p]) + b[p]
            return carry

        lax.fori_loop(0, _CH, row_body, 0, unroll=2)

    fire_chunk(0, 0)

    def chunk_loop(c, _):
        s = lax.rem(c, _NSLOT)
        drain_gathers(s)

        @pl.when(c + 1 < _NCHUNK)
        def _prefetch():
            ns = lax.rem(c + 1, _NSLOT)

            @pl.when(c >= _NSLOT - 1)
            def _wait_out():
                drain_out(ns)

            fire_chunk(c + 1, ns)

        compute_chunk(s)
        fire_out(c, s)
        return 0

    lax.fori_loop(0, _NCHUNK, chunk_loop, 0)

    for k in range(_NSLOT):
        drain_out(lax.rem(jnp.int32(_NCHUNK - _NSLOT + k), _NSLOT))


@jax.jit
def _lru_embed_sc(xf, table, gamma, beta):
    mesh = plsc.VectorSubcoreMesh(core_axis_name="c", subcore_axis_name="s",
                                  num_cores=_NC, num_subcores=_NS)
    return pl.kernel(
        _sc_body,
        out_type=(
            jax.ShapeDtypeStruct((2 * _N, _EMBED), jnp.float32),
            jax.ShapeDtypeStruct((_N,), jnp.int32),
        ),
        mesh=mesh,
        compiler_params=pltpu.CompilerParams(use_tc_tiling_on_sc=False,
                                             needs_layout_passes=False),
        scratch_types=[
            pltpu.VMEM((_NSLOT, _CH), jnp.int32),
            pltpu.VMEM((_NSLOT, 2 * _CH), jnp.int32),
            pltpu.VMEM((_NSLOT, 2 * _CH, _EMBED), jnp.float32),
            pltpu.VMEM((_NSLOT, _CH), jnp.int32),
            pltpu.VMEM((2, _EMBED), jnp.float32),
            pltpu.SemaphoreType.DMA((_NSLOT,)),
            pltpu.SemaphoreType.DMA((_NSLOT,)),
        ],
    )(xf, table, gamma, beta)


def kernel(x, table, ln_gamma, ln_beta):
    xf = x.reshape(_N)
    outp, mask_i32 = _lru_embed_sc(xf, table, ln_gamma, ln_beta)
    out = outp.reshape(_B, _L, 128)[:, :, :_EMBED]
    mask = mask_i32.reshape(_B, _L).astype(jnp.bool_)
    return out, mask


# speedup vs baseline: 1.0593x; 1.0593x over previous
import jax
import jax.numpy as jnp
from jax import lax
from jax.experimental import pallas as pl
from jax.experimental.pallas import tpu as pltpu
from jax.experimental.pallas import tpu_sc as plsc

_VOCAB = 100000
_EMBED = 64
_B = 4096
_L = 200

_NC = 2
_NS = 16
_NW = _NC * _NS
_LANES = 16

_N = _B * _L
_RPW = _N // _NW
_CH = 256
_NCHUNK = _RPW // _CH
_NSLOT = 3
_NWIN = (2 * _CH) // 128


def _sc_body(x_hbm, table_hbm, gamma_hbm, beta_hbm, out_hbm, mask_hbm,
             xt_v, idx_v, rows_v, mask_v, gb_v, sem_g, sem_o):
    wid = lax.axis_index("s") * _NC + lax.axis_index("c")

    pltpu.sync_copy(gamma_hbm, gb_v.at[0])
    pltpu.sync_copy(beta_hbm, gb_v.at[1])
    g = [gb_v[0, pl.ds(16 * p, 16)] for p in range(4)]
    b = [gb_v[1, pl.ds(16 * p, 16)] for p in range(4)]

    one = jnp.full((_LANES,), 1, dtype=jnp.int32)
    zero = jnp.full((_LANES,), 0, dtype=jnp.int32)
    lane = lax.iota(jnp.int32, _LANES)
    evens = lane * 2

    perms = [lax.bitwise_xor(lane, jnp.int32(k)) for k in (8, 4, 2, 1)]
    dnums = lax.GatherDimensionNumbers(
        offset_dims=(), collapsed_slice_dims=(0,), start_index_map=(0,))

    def allsum(vec):
        for p in perms:
            vec = vec + lax.gather(
                vec, p[:, None], dnums, slice_sizes=(1,),
                mode=lax.GatherScatterMode.PROMISE_IN_BOUNDS)
        return vec

    def rsqrt_vec(x):
        i = lax.bitcast_convert_type(x, jnp.int32)
        i = jnp.int32(0x5F3759DF) - lax.shift_right_logical(i, 1)
        y = lax.bitcast_convert_type(i, jnp.float32)
        xh = x * jnp.float32(0.5)
        for _ in range(2):
            y = y * (jnp.float32(1.5) - xh * y * y)
        return y

    def fire_chunk(c, s):
        fbase = wid * _RPW + c * _CH
        pltpu.sync_copy(x_hbm.at[pl.ds(fbase, _CH)], xt_v.at[s])
        islot = idx_v.at[s]
        for t in range(_CH // 16):
            iv = xt_v[s, pl.ds(16 * t, 16)]
            pos = evens + (32 * t)
            plsc.store_scatter(islot, [pos], iv)
            plsc.store_scatter(islot, [pos + 1], iv)
        for w in range(_NWIN):
            pltpu.async_copy(table_hbm.at[idx_v.at[s, pl.ds(w * 128, 128)]],
                             rows_v.at[s, pl.ds(w * 128, 128)], sem_g.at[s])

    def drain_gathers(s):
        for w in range(_NWIN):
            pltpu.make_async_copy(
                table_hbm.at[idx_v.at[s, pl.ds(w * 128, 128)]],
                rows_v.at[s, pl.ds(w * 128, 128)], sem_g.at[s]).wait()

    def fire_out(c, s):
        base = wid * _RPW + c * _CH
        pltpu.async_copy(rows_v.at[s], out_hbm.at[pl.ds(2 * base, 2 * _CH)],
                         sem_o.at[s])
        pltpu.async_copy(mask_v.at[s], mask_hbm.at[pl.ds(base, _CH)],
                         sem_o.at[s])

    def drain_out(s):
        pltpu.make_async_copy(rows_v.at[s], out_hbm.at[pl.ds(0, 2 * _CH)],
                              sem_o.at[s]).wait()
        pltpu.make_async_copy(mask_v.at[s], mask_hbm.at[pl.ds(0, _CH)],
                              sem_o.at[s]).wait()

    def compute_chunk(s):
        for t in range(_CH // 16):
            iv = xt_v[s, pl.ds(16 * t, 16)]
            mask_v[s, pl.ds(16 * t, 16)] = jnp.where(iv > 0, one, zero)

        rslot = rows_v.at[s]

        def row_body(r, carry):
            r2 = r * 2
            v = [rslot[r2, pl.ds(16 * p, 16)] for p in range(4)]
            ssum = (v[0] + v[1]) + (v[2] + v[3])
            sq = (v[0] * v[0] + v[1] * v[1]) + (v[2] * v[2] + v[3] * v[3])
            mean = allsum(ssum) * jnp.float32(1.0 / 64.0)
            var = allsum(sq) * jnp.float32(1.0 / 64.0) - mean * mean
            inv = rsqrt_vec(var + jnp.float32(1e-5))
            for p in range(4):
                rslot[r2, pl.ds(16 * p, 16)] = \
                    (v[p] - mean) * (inv * g[p]) + b[p]
            return carry

        lax.fori_loop(0, _CH, row_body, 0, unroll=2)

    fire_chunk(0, 0)

    def chunk_loop(c, _):
        s = lax.rem(c, _NSLOT)
        drain_gathers(s)

        @pl.when(c + 1 < _NCHUNK)
        def _prefetch():
            ns = lax.rem(c + 1, _NSLOT)

            @pl.when(c >= _NSLOT - 1)
            def _wait_out():
                drain_out(ns)

            fire_chunk(c + 1, ns)

        compute_chunk(s)
        fire_out(c, s)
        return 0

    lax.fori_loop(0, _NCHUNK, chunk_loop, 0)

    for k in range(_NSLOT):
        drain_out(lax.rem(jnp.int32(_NCHUNK - _NSLOT + k), _NSLOT))


@jax.jit
def _lru_embed_sc(xf, table, gamma, beta):
    mesh = plsc.VectorSubcoreMesh(core_axis_name="c", subcore_axis_name="s",
                                  num_cores=_NC, num_subcores=_NS)
    return pl.kernel(
        _sc_body,
        out_type=(
            jax.ShapeDtypeStruct((2 * _N, _EMBED), jnp.float32),
            jax.ShapeDtypeStruct((_N,), jnp.int32),
        ),
        mesh=mesh,
        compiler_params=pltpu.CompilerParams(use_tc_tiling_on_sc=False,
                                             needs_layout_passes=False),
        scratch_types=[
            pltpu.VMEM((_NSLOT, _CH), jnp.int32),
            pltpu.VMEM((_NSLOT, 2 * _CH), jnp.int32),
            pltpu.VMEM((_NSLOT, 2 * _CH, _EMBED), jnp.float32),
            pltpu.VMEM((_NSLOT, _CH), jnp.int32),
            pltpu.VMEM((2, _EMBED), jnp.float32),
            pltpu.SemaphoreType.DMA((_NSLOT,)),
            pltpu.SemaphoreType.DMA((_NSLOT,)),
        ],
    )(xf, table, gamma, beta)


def kernel(x, table, ln_gamma, ln_beta):
    xf = x.reshape(_N)
    outp, mask_i32 = _lru_embed_sc(xf, table, ln_gamma, ln_beta)
    out = outp.reshape(_B, _L, 128)[:, :, :_EMBED]
    mask = mask_i32.reshape(_B, _L).astype(jnp.bool_)
    return out, mask
